# baseline (speedup 1.0000x reference)
import functools

import jax
import jax.numpy as jnp
from jax import lax
from jax.experimental import pallas as pl
from jax.experimental.pallas import tpu as pltpu, tpu_sc as plsc

N = 10000
NN = 10240
NW = 32
RPT = NN // NW
GB = 128
OUT = 64
CLS = 16
Hh = Ww = 384
HP = WP = 386
NPIX = Hh * Ww
RPAD = 151552

_SC_PARAMS = pltpu.CompilerParams(use_tc_tiling_on_sc=False,
                                  needs_layout_passes=False)


def _wid():
    return lax.axis_index("s") * 2 + lax.axis_index("c")


def _make_seg_mean(NP, W, CH):
    nch = NP // CH
    nvr = CH // 16
    mesh = plsc.VectorSubcoreMesh(core_axis_name="c", subcore_axis_name="s")

    @functools.partial(
        pl.kernel,
        out_type=jax.ShapeDtypeStruct((NN, W), jnp.float32),
        mesh=mesh,
        compiler_params=_SC_PARAMS,
        scratch_types=[
            pltpu.VMEM((RPT, W), jnp.float32),
            pltpu.VMEM((RPT, 16), jnp.float32),
            pltpu.VMEM((2, CH), jnp.int32),
            pltpu.VMEM((CH + 16,), jnp.int32),
            pltpu.VMEM((CH + 16,), jnp.int32),
            pltpu.VMEM((2, GB, W), jnp.float32),
            pltpu.SemaphoreType.DMA,
            pltpu.SemaphoreType.DMA,
        ],
    )
    def k(x_hbm, asn_hbm, out_hbm, acc, cnt, asnb, midx, mloc, gbuf,
          csem, gsem):
        lo = _wid() * RPT
        zf = jnp.zeros((16,), jnp.float32)

        def init_row(r, _):
            for kk in range(W // 16):
                acc[r, pl.ds(kk * 16, 16)] = zf
            cnt[r, pl.ds(0, 16)] = zf
            return 0
        lax.fori_loop(0, RPT, init_row, 0)

        def zero16(i, _):
            midx[pl.ds(i * 16, 16)] = jnp.zeros((16,), jnp.int32)
            return 0
        lax.fori_loop(0, (CH + 16) // 16, zero16, 0)

        def start_chunk(ci):
            pltpu.async_copy(asn_hbm.at[pl.ds(ci * CH, CH)],
                             asnb.at[lax.rem(ci, 2)], csem)

        start_chunk(0)
        lane = lax.iota(jnp.int32, 16)
        ones = jnp.ones((16,), jnp.float32)

        def chunk(ci, _):
            slot = lax.rem(ci, 2)
            pltpu.make_async_copy(asn_hbm.at[pl.ds(0, CH)], asnb.at[slot],
                                  csem).wait()

            @pl.when(ci + 1 < nch)
            def _():
                start_chunk(ci + 1)

            def scan(v, cnt_c):
                a = asnb[slot, pl.ds(v * 16, 16)]
                al = a - lo
                m = (al >= 0) & (al < RPT)
                pid = (ci * CH + v * 16) + lane
                pos = plsc.cumsum(m.astype(jnp.int32))
                tgt = cnt_c + pos - 1
                plsc.store_scatter(midx, [tgt], pid, mask=m)
                plsc.store_scatter(mloc, [tgt], al, mask=m)
                return cnt_c + pos[15]

            mtot = lax.fori_loop(0, nvr, scan, jnp.int32(0))
            nb = lax.div(mtot + (GB - 1), GB)

            def start_g(g):
                pltpu.async_copy(x_hbm.at[midx.at[pl.ds(g * GB, GB)]],
                                 gbuf.at[lax.rem(g, 2)], gsem)

            @pl.when(nb > 0)
            def _():
                start_g(0)

            def batch(g, _):
                gslot = lax.rem(g, 2)
                pltpu.make_async_copy(x_hbm.at[midx.at[pl.ds(0, GB)]],
                                      gbuf.at[gslot], gsem).wait()

                @pl.when(g + 1 < nb)
                def _():
                    start_g(g + 1)

                bs = jnp.minimum(mtot - g * GB, GB)

                def rmw(i, _):
                    l = mloc[pl.ds(g * GB + i, 16)][0]
                    for kk in range(W // 16):
                        sl = pl.ds(kk * 16, 16)
                        acc[l, sl] = acc[l, sl] + gbuf[gslot, i, sl]
                    cnt[l, pl.ds(0, 16)] = cnt[l, pl.ds(0, 16)] + ones
                    return 0
                lax.fori_loop(0, bs, rmw, 0)
                return 0

            lax.fori_loop(0, nb, batch, 0)
            return 0

        lax.fori_loop(0, nch, chunk, 0)

        def finish(r, _):
            c = jnp.maximum(cnt[r, pl.ds(0, 16)][0], 1.0)
            inv = jnp.full((16,), 1.0, jnp.float32) / c
            for kk in range(W // 16):
                sl = pl.ds(kk * 16, 16)
                acc[r, sl] = acc[r, sl] * inv
            return 0
        lax.fori_loop(0, RPT, finish, 0)
        pltpu.sync_copy(acc, out_hbm.at[pl.ds(lo, RPT)])

    return k


def _make_edge_max(E, W, CH, n_extra):
    nch = E // CH
    nvr = CH // 16
    mesh = plsc.VectorSubcoreMesh(core_axis_name="c", subcore_axis_name="s")

    @functools.partial(
        pl.kernel,
        out_type=jax.ShapeDtypeStruct((NN, W), jnp.float32),
        mesh=mesh,
        compiler_params=_SC_PARAMS,
        scratch_types=[
            pltpu.VMEM((RPT, W), jnp.float32),
            pltpu.VMEM((RPT, W), jnp.float32),
            pltpu.VMEM((2, CH), jnp.int32),
            pltpu.VMEM((2, CH), jnp.int32),
            pltpu.VMEM((CH + 16,), jnp.int32),
            pltpu.VMEM((CH + 16,), jnp.int32),
            pltpu.VMEM((2, GB, W), jnp.float32),
            pltpu.VMEM((W,), jnp.float32),
            pltpu.VMEM((max(n_extra, 1), RPT, W), jnp.float32),
            pltpu.SemaphoreType.DMA,
            pltpu.SemaphoreType.DMA,
            pltpu.SemaphoreType.DMA,
        ],
    )
    def k(p_hbm, q_hbm, src_hbm, dst_hbm, b_hbm, *rest):
        extras_hbm = rest[:n_extra]
        out_hbm = rest[n_extra]
        (acc, pbuf, dstb, srcb, midx, mloc, gbuf, bvec, ebuf,
         csem, gsem, psem) = rest[n_extra + 1:]
        lo = _wid() * RPT

        pltpu.async_copy(p_hbm.at[pl.ds(lo, RPT)], pbuf, psem)
        pltpu.async_copy(b_hbm, bvec, psem)
        for t in range(n_extra):
            pltpu.async_copy(extras_hbm[t].at[pl.ds(lo, RPT)], ebuf.at[t],
                             psem)

        neg = jnp.full((16,), -jnp.inf, jnp.float32)

        def init_row(r, _):
            for kk in range(W // 16):
                acc[r, pl.ds(kk * 16, 16)] = neg
            return 0
        lax.fori_loop(0, RPT, init_row, 0)

        def zero16(i, _):
            midx[pl.ds(i * 16, 16)] = jnp.zeros((16,), jnp.int32)
            return 0
        lax.fori_loop(0, (CH + 16) // 16, zero16, 0)

        def start_chunk(ci):
            slot = lax.rem(ci, 2)
            pltpu.async_copy(dst_hbm.at[pl.ds(ci * CH, CH)], dstb.at[slot],
                             csem)
            pltpu.async_copy(src_hbm.at[pl.ds(ci * CH, CH)], srcb.at[slot],
                             csem)

        start_chunk(0)

        def chunk(ci, _):
            slot = lax.rem(ci, 2)
            pltpu.make_async_copy(dst_hbm.at[pl.ds(0, CH)], dstb.at[slot],
                                  csem).wait()
            pltpu.make_async_copy(src_hbm.at[pl.ds(0, CH)], srcb.at[slot],
                                  csem).wait()

            @pl.when(ci + 1 < nch)
            def _():
                start_chunk(ci + 1)

            def scan(v, cnt):
                d = dstb[slot, pl.ds(v * 16, 16)]
                s = srcb[slot, pl.ds(v * 16, 16)]
                dl = d - lo
                m = (dl >= 0) & (dl < RPT)
                pos = plsc.cumsum(m.astype(jnp.int32))
                tgt = cnt + pos - 1
                plsc.store_scatter(midx, [tgt], s, mask=m)
                plsc.store_scatter(mloc, [tgt], dl, mask=m)
                return cnt + pos[15]

            mtot = lax.fori_loop(0, nvr, scan, jnp.int32(0))
            nb = lax.div(mtot + (GB - 1), GB)

            def start_g(g):
                pltpu.async_copy(q_hbm.at[midx.at[pl.ds(g * GB, GB)]],
                                 gbuf.at[lax.rem(g, 2)], gsem)

            @pl.when(nb > 0)
            def _():
                start_g(0)

            def batch(g, _):
                gslot = lax.rem(g, 2)
                pltpu.make_async_copy(q_hbm.at[midx.at[pl.ds(0, GB)]],
                                      gbuf.at[gslot], gsem).wait()

                @pl.when(g + 1 < nb)
                def _():
                    start_g(g + 1)

                bs = jnp.minimum(mtot - g * GB, GB)

                def rmw(i, _):
                    l = mloc[pl.ds(g * GB + i, 16)][0]
                    for kk in range(W // 16):
                        sl = pl.ds(kk * 16, 16)
                        acc[l, sl] = jnp.maximum(acc[l, sl],
                                                 gbuf[gslot, i, sl])
                    return 0
                lax.fori_loop(0, bs, rmw, 0)
                return 0

            lax.fori_loop(0, nb, batch, 0)
            return 0

        lax.fori_loop(0, nch, chunk, 0)

        pltpu.make_async_copy(p_hbm.at[pl.ds(lo, RPT)], pbuf, psem).wait()
        pltpu.make_async_copy(b_hbm, bvec, psem).wait()
        for t in range(n_extra):
            pltpu.make_async_copy(extras_hbm[t].at[pl.ds(lo, RPT)],
                                  ebuf.at[t], psem).wait()

        def comb(r, _):
            for kk in range(W // 16):
                sl = pl.ds(kk * 16, 16)
                h = jnp.maximum(acc[r, sl] + pbuf[r, sl] + bvec[sl], 0.0)
                for t in range(n_extra):
                    h = h + ebuf[t, r, sl]
                pbuf[r, sl] = h
            return 0
        lax.fori_loop(0, RPT, comb, 0)
        pltpu.sync_copy(pbuf, out_hbm.at[pl.ds(lo, RPT)])

    return k


def _make_pix_gather(R, D, NBUF=8):
    per = R // NW
    nch = per // GB
    mesh = plsc.VectorSubcoreMesh(core_axis_name="c", subcore_axis_name="s")

    @functools.partial(
        pl.kernel,
        out_type=jax.ShapeDtypeStruct((R, D), jnp.float32),
        mesh=mesh,
        compiler_params=_SC_PARAMS,
        scratch_types=[
            pltpu.VMEM((per,), jnp.int32),
            pltpu.VMEM((NBUF, GB, D), jnp.float32),
            pltpu.SemaphoreType.DMA,
            pltpu.SemaphoreType.DMA,
        ],
    )
    def k(table_hbm, idx_hbm, out_hbm, idx_v, bufs, gsem, osem):
        base = _wid() * per
        pltpu.sync_copy(idx_hbm.at[pl.ds(base, per)], idx_v)

        def start_gather(ch, slot):
            pltpu.async_copy(table_hbm.at[idx_v.at[pl.ds(ch * GB, GB)]],
                             bufs.at[slot], gsem)

        for s in range(NBUF):
            start_gather(s, s)

        def step(ch, _):
            slot = lax.rem(ch, NBUF)
            pltpu.make_async_copy(table_hbm.at[idx_v.at[pl.ds(0, GB)]],
                                  bufs.at[slot], gsem).wait()
            pltpu.async_copy(bufs.at[slot],
                             out_hbm.at[pl.ds(base + ch * GB, GB)],
                             osem).wait()

            @pl.when(ch + NBUF < nch)
            def _():
                start_gather(ch + NBUF, slot)
            return 0

        lax.fori_loop(0, nch, step, 0)

    return k


def _mm_body(x_ref, w_ref, p_ref, q_ref):
    r = lax.dot_general(x_ref[...], w_ref[...], (((1,), (0,)), ((), ())),
                        preferred_element_type=jnp.float32)
    p_ref[...] = r[:, 0:OUT]
    q_ref[...] = r[:, OUT:2 * OUT]


def _mm(x, wc):
    return pl.pallas_call(
        _mm_body,
        out_shape=(jax.ShapeDtypeStruct((NN, OUT), jnp.float32),
                   jax.ShapeDtypeStruct((NN, OUT), jnp.float32)),
        in_specs=[pl.BlockSpec(memory_space=pltpu.VMEM),
                  pl.BlockSpec(memory_space=pltpu.VMEM)],
        out_specs=(pl.BlockSpec(memory_space=pltpu.VMEM),
                   pl.BlockSpec(memory_space=pltpu.VMEM)),
    )(x, wc)


def _conv_lin_body(img_ref, w_ref, bias_ref, out_ref, buf, obuf, sem, osem):
    def blk(i, _):
        cp = pltpu.make_async_copy(
            img_ref.at[pl.ds(i * 64 * WP, 66 * WP)], buf, sem)
        cp.start()
        cp.wait()

        def sub(j, _):
            g0 = j * 16 * WP
            a = buf[pl.ds(g0, 16 * WP)]
            b = buf[pl.ds(g0 + WP, 16 * WP)]
            c = buf[pl.ds(g0 + 2 * WP, 16 * WP)]
            u = jnp.concatenate([a, b, c], axis=-1)
            r = lax.dot_general(u, w_ref[...], (((1,), (0,)), ((), ())),
                                preferred_element_type=jnp.float32)
            r = r.reshape(16, WP, 3 * CLS)
            obuf[...] = (r[:, 0:Ww, 0:CLS] + r[:, 1:Ww + 1, CLS:2 * CLS]
                         + r[:, 2:Ww + 2, 2 * CLS:3 * CLS] + bias_ref[0:CLS])
            ocp = pltpu.make_async_copy(
                obuf, out_ref.at[pl.ds(i * 64 + j * 16, 16)], osem)
            ocp.start()
            ocp.wait()
            return 0

        lax.fori_loop(0, 4, sub, 0)
        return 0

    lax.fori_loop(0, 6, blk, 0)


def _conv_lin(img_flat, wcat, bias):
    return pl.pallas_call(
        _conv_lin_body,
        out_shape=jax.ShapeDtypeStruct((Hh, Ww, CLS), jnp.float32),
        in_specs=[
            pl.BlockSpec(memory_space=pl.ANY),
            pl.BlockSpec(memory_space=pltpu.VMEM),
            pl.BlockSpec(memory_space=pltpu.VMEM),
        ],
        out_specs=pl.BlockSpec(memory_space=pl.ANY),
        compiler_params=pltpu.CompilerParams(
            vmem_limit_bytes=60 * 1024 * 1024),
        scratch_shapes=[
            pltpu.VMEM((66 * WP, OUT), jnp.float32),
            pltpu.VMEM((16, Ww, CLS), jnp.float32),
            pltpu.SemaphoreType.DMA,
            pltpu.SemaphoreType.DMA,
        ],
    )(img_flat, wcat, bias)


def kernel(x, edge_index, assign, W1, b1, W2, b2, conv_w, conv_b, lin_w,
           lin_b):
    h, w, c = x.shape
    x_flat = x.reshape(h * w, c)
    src, dst = edge_index[0], edge_index[1]

    sp = _make_seg_mean(NPIX, 128, 4096)(x_flat, assign)

    wc1 = jnp.concatenate([W1[:c] - W1[c:], W1[c:]], axis=1)
    wc2 = jnp.concatenate([W2[:OUT] - W2[OUT:], W2[OUT:]], axis=1)

    edge1 = _make_edge_max(160000, OUT, 4000, 0)
    edge3 = _make_edge_max(160000, OUT, 4000, 2)

    p1, q1 = _mm(sp, wc1)
    h1 = edge1(p1, q1, src, dst, b1)
    p2, q2 = _mm(h1, wc2)
    h2 = edge1(p2, q2, src, dst, b2)
    p3, q3 = _mm(h2, wc2)
    h4 = edge3(p3, q3, src, dst, b2, h1, h2)

    yy, xx = jnp.meshgrid(jnp.arange(HP), jnp.arange(WP), indexing="ij")
    interior = (yy >= 1) & (yy <= Hh) & (xx >= 1) & (xx <= Ww)
    pix = (yy - 1) * Ww + (xx - 1)
    apad = jnp.where(interior, assign[jnp.clip(pix, 0, NPIX - 1)], N)
    apad = jnp.concatenate(
        [apad.reshape(-1),
         jnp.full((RPAD - HP * WP,), N, jnp.int32)]).astype(jnp.int32)

    img_flat = _make_pix_gather(RPAD, OUT)(h4, apad)

    m = jnp.einsum("oikl,oc->klic", conv_w, lin_w)
    wcat = jnp.concatenate(
        [jnp.concatenate([m[ky, kx] for ky in range(3)], axis=0)
         for kx in range(3)], axis=1)
    bias = jnp.tile(conv_b @ lin_w + lin_b, 3)

    out = _conv_lin(img_flat, wcat, bias)
    return out.reshape(Hh * Ww, CLS)
